# baseline (speedup 1.0000x reference)
import functools
import math

import jax
import jax.numpy as jnp
from jax.experimental import pallas as pl

N = 8192
D = 512
KK = 33
BM = 256
BITERS = 40


def _emb_body(f_ref, w0_ref, w1_ref, emb_ref):
    h = jnp.maximum(f_ref[...] * w0_ref[...], 0.0) * w1_ref[...]
    nrm = jnp.sqrt(jnp.sum(h * h, axis=1, keepdims=True))
    emb_ref[...] = h / jnp.maximum(nrm, 1e-12)


def _topk_body(emb_blk_ref, emb_all_ref, out_ref, *, kk, biters):
    bm = emb_blk_ref.shape[0]
    n = emb_all_ref.shape[0]
    sim = jax.lax.dot_general(
        emb_blk_ref[...], emb_all_ref[...],
        (((1,), (1,)), ((), ())),
        preferred_element_type=jnp.float32,
    )
    kkf = float(kk)
    lt = math.log(kkf + 1.0)
    niterp = 8

    def cond(state):
        i, _, _, cl, _ = state
        return jnp.logical_and(i < biters, jnp.any(cl != kkf))

    def step(state):
        i, lo, hi, cl, ch = state
        frac_i = (jnp.log(cl + 1.0) - lt) / jnp.maximum(
            jnp.log(cl + 1.0) - jnp.log(ch + 1.0), 1e-9)
        frac = jnp.where(i < niterp,
                         jnp.clip(frac_i, 0.05, 0.95),
                         jnp.full_like(frac_i, 0.5))
        mid = lo + (hi - lo) * frac
        cnt = jnp.sum(jnp.where(sim >= mid, 1.0, 0.0), axis=1, keepdims=True)
        pred = cnt >= kkf
        return (i + 1, jnp.where(pred, mid, lo), jnp.where(pred, hi, mid),
                jnp.where(pred, cnt, cl), jnp.where(pred, ch, cnt))

    def step3(state):
        return step(step(step(state)))

    lo0 = jnp.full((bm, 1), -1.001, jnp.float32)
    hi0 = jnp.full((bm, 1), 1.001, jnp.float32)
    cl0 = jnp.full((bm, 1), float(n), jnp.float32)
    ch0 = jnp.zeros((bm, 1), jnp.float32)
    _, lo, _, _, _ = jax.lax.while_loop(cond, step3, (0, lo0, hi0, cl0, ch0))
    out_ref[...] = jnp.where(sim >= lo, jnp.maximum(sim, 0.0), 0.0)


def _build(n, d, bm, kk, biters, interpret=False):
    emb_call = pl.pallas_call(
        _emb_body,
        grid=(8,),
        in_specs=[
            pl.BlockSpec((n // 8, d), lambda i: (i, 0)),
            pl.BlockSpec((1, d), lambda i: (0, 0)),
            pl.BlockSpec((1, d), lambda i: (0, 0)),
        ],
        out_specs=pl.BlockSpec((n // 8, d), lambda i: (i, 0)),
        out_shape=jax.ShapeDtypeStruct((n, d), jnp.float32),
        interpret=interpret,
    )
    topk_call = pl.pallas_call(
        functools.partial(_topk_body, kk=kk, biters=biters),
        grid=(n // bm,),
        in_specs=[
            pl.BlockSpec((bm, d), lambda i: (i, 0)),
            pl.BlockSpec((n, d), lambda i: (0, 0)),
        ],
        out_specs=pl.BlockSpec((bm, n), lambda i: (i, 0)),
        out_shape=jax.ShapeDtypeStruct((n, n), jnp.float32),
        interpret=interpret,
    )
    return emb_call, topk_call


def kernel(features, w0, w1):
    n, d = features.shape
    emb_call, topk_call = _build(n, d, BM, KK, BITERS)
    emb = emb_call(features, w0.reshape(1, d), w1.reshape(1, d))
    return topk_call(emb, emb)


# speedup vs baseline: 1.0071x; 1.0071x over previous
import functools
import math

import jax
import jax.numpy as jnp
from jax.experimental import pallas as pl

N = 8192
D = 512
KK = 33
BM = 256
BITERS = 40


def _emb_body(f_ref, w0_ref, w1_ref, emb_ref):
    h = jnp.maximum(f_ref[...] * w0_ref[...], 0.0) * w1_ref[...]
    nrm = jnp.sqrt(jnp.sum(h * h, axis=1, keepdims=True))
    emb_ref[...] = (h / jnp.maximum(nrm, 1e-12)).astype(jnp.bfloat16)


def _topk_body(emb_blk_ref, emb_all_ref, out_ref, *, kk, biters):
    bm = emb_blk_ref.shape[0]
    n = emb_all_ref.shape[0]
    sim = jax.lax.dot_general(
        emb_blk_ref[...], emb_all_ref[...],
        (((1,), (1,)), ((), ())),
        preferred_element_type=jnp.float32,
    )
    kkf = float(kk)
    lt = math.log(kkf + 1.0)
    niterp = 8

    def cond(state):
        i, _, _, cl, _ = state
        return jnp.logical_and(i < biters, jnp.any(cl != kkf))

    def step(state):
        i, lo, hi, cl, ch = state
        frac_i = (jnp.log(cl + 1.0) - lt) / jnp.maximum(
            jnp.log(cl + 1.0) - jnp.log(ch + 1.0), 1e-9)
        frac = jnp.where(i < niterp,
                         jnp.clip(frac_i, 0.05, 0.95),
                         jnp.full_like(frac_i, 0.5))
        mid = lo + (hi - lo) * frac
        cnt = jnp.sum(jnp.where(sim >= mid, 1.0, 0.0), axis=1, keepdims=True)
        pred = cnt >= kkf
        return (i + 1, jnp.where(pred, mid, lo), jnp.where(pred, hi, mid),
                jnp.where(pred, cnt, cl), jnp.where(pred, ch, cnt))

    def step3(state):
        return step(step(step(state)))

    lo0 = jnp.full((bm, 1), -1.001, jnp.float32)
    hi0 = jnp.full((bm, 1), 1.001, jnp.float32)
    cl0 = jnp.full((bm, 1), float(n), jnp.float32)
    ch0 = jnp.zeros((bm, 1), jnp.float32)
    _, lo, _, _, _ = jax.lax.while_loop(cond, step3, (0, lo0, hi0, cl0, ch0))
    out_ref[...] = jnp.where(sim >= lo, jnp.maximum(sim, 0.0), 0.0)


def _build(n, d, bm, kk, biters, interpret=False):
    emb_call = pl.pallas_call(
        _emb_body,
        grid=(8,),
        in_specs=[
            pl.BlockSpec((n // 8, d), lambda i: (i, 0)),
            pl.BlockSpec((1, d), lambda i: (0, 0)),
            pl.BlockSpec((1, d), lambda i: (0, 0)),
        ],
        out_specs=pl.BlockSpec((n // 8, d), lambda i: (i, 0)),
        out_shape=jax.ShapeDtypeStruct((n, d), jnp.bfloat16),
        interpret=interpret,
    )
    topk_call = pl.pallas_call(
        functools.partial(_topk_body, kk=kk, biters=biters),
        grid=(n // bm,),
        in_specs=[
            pl.BlockSpec((bm, d), lambda i: (i, 0)),
            pl.BlockSpec((n, d), lambda i: (0, 0)),
        ],
        out_specs=pl.BlockSpec((bm, n), lambda i: (i, 0)),
        out_shape=jax.ShapeDtypeStruct((n, n), jnp.float32),
        interpret=interpret,
    )
    return emb_call, topk_call


def kernel(features, w0, w1):
    n, d = features.shape
    emb_call, topk_call = _build(n, d, BM, KK, BITERS)
    emb = emb_call(features, w0.reshape(1, d), w1.reshape(1, d))
    return topk_call(emb, emb)


# speedup vs baseline: 1.1182x; 1.1103x over previous
import functools
import math

import jax
import jax.numpy as jnp
from jax.experimental import pallas as pl

N = 8192
D = 512
KK = 33
BM = 256
BITERS = 40


def _emb_body(f_ref, w0_ref, w1_ref, emb_ref):
    h = jnp.maximum(f_ref[...] * w0_ref[...], 0.0) * w1_ref[...]
    nrm = jnp.sqrt(jnp.sum(h * h, axis=1, keepdims=True))
    emb_ref[...] = (h / jnp.maximum(nrm, 1e-12)).astype(jnp.bfloat16)


def _topk_body(emb_blk_ref, emb_all_ref, out_ref, *, kk, biters):
    bm = emb_blk_ref.shape[0]
    n = emb_all_ref.shape[0]
    sim = jax.lax.dot_general(
        emb_blk_ref[...], emb_all_ref[...],
        (((1,), (1,)), ((), ())),
        preferred_element_type=jnp.float32,
    )
    kkf = float(kk)
    lt = math.log(kkf + 1.0)
    niterp = 16

    def cond(state):
        i, _, _, cl, _ = state
        return jnp.logical_and(i < biters, jnp.any(cl != kkf))

    def step(state):
        i, lo, hi, cl, ch = state
        frac_i = (jnp.log(cl + 1.0) - lt) / jnp.maximum(
            jnp.log(cl + 1.0) - jnp.log(ch + 1.0), 1e-9)
        frac = jnp.where(i < niterp,
                         jnp.clip(frac_i, 0.05, 0.95),
                         jnp.full_like(frac_i, 0.5))
        mid = lo + (hi - lo) * frac
        cnt = jnp.sum(jnp.where(sim >= mid, 1.0, 0.0), axis=1, keepdims=True)
        pred = cnt >= kkf
        return (i + 1, jnp.where(pred, mid, lo), jnp.where(pred, hi, mid),
                jnp.where(pred, cnt, cl), jnp.where(pred, ch, cnt))

    def step3(state):
        return step(step(step(state)))

    lo0 = jnp.full((bm, 1), -1.001, jnp.float32)
    hi0 = jnp.full((bm, 1), 1.001, jnp.float32)
    cl0 = jnp.full((bm, 1), float(n), jnp.float32)
    ch0 = jnp.zeros((bm, 1), jnp.float32)
    _, lo, _, _, _ = jax.lax.while_loop(cond, step3, (0, lo0, hi0, cl0, ch0))
    out_ref[...] = jnp.where(sim >= lo, jnp.maximum(sim, 0.0), 0.0)


def _build(n, d, bm, kk, biters, interpret=False):
    emb_call = pl.pallas_call(
        _emb_body,
        grid=(8,),
        in_specs=[
            pl.BlockSpec((n // 8, d), lambda i: (i, 0)),
            pl.BlockSpec((1, d), lambda i: (0, 0)),
            pl.BlockSpec((1, d), lambda i: (0, 0)),
        ],
        out_specs=pl.BlockSpec((n // 8, d), lambda i: (i, 0)),
        out_shape=jax.ShapeDtypeStruct((n, d), jnp.bfloat16),
        interpret=interpret,
    )
    topk_call = pl.pallas_call(
        functools.partial(_topk_body, kk=kk, biters=biters),
        grid=(n // bm,),
        in_specs=[
            pl.BlockSpec((bm, d), lambda i: (i, 0)),
            pl.BlockSpec((n, d), lambda i: (0, 0)),
        ],
        out_specs=pl.BlockSpec((bm, n), lambda i: (i, 0)),
        out_shape=jax.ShapeDtypeStruct((n, n), jnp.float32),
        interpret=interpret,
    )
    return emb_call, topk_call


def kernel(features, w0, w1):
    n, d = features.shape
    emb_call, topk_call = _build(n, d, BM, KK, BITERS)
    emb = emb_call(features, w0.reshape(1, d), w1.reshape(1, d))
    return topk_call(emb, emb)
